# baseline (speedup 1.0000x reference)
import functools

import jax
import jax.numpy as jnp
from jax import lax
from jax.experimental import pallas as pl
from jax.experimental.pallas import tpu as pltpu
from jax.experimental.pallas import tpu_sc as plsc

N0 = 10000
E0 = 320000
EH0 = 100000
NHE0 = 2000
DF = 128
DIM = 64
DIMP = 128
NCLS = 40

NCORE = 2
NSUB = 16
NW = NCORE * NSUB
C = 128

NR = 10240
BH = 2048

KE = 80
KH = 26
EP = NW * KE * C
EHP = NW * KH * C

_MESH = plsc.VectorSubcoreMesh(
    core_axis_name="c", subcore_axis_name="s",
    num_cores=NCORE, num_subcores=NSUB)


def _fill_zero_rows(buf):
    z = jnp.zeros((16,), jnp.float32)

    @pl.loop(0, C)
    def _(r):
        for j in range(DIMP // 16):
            buf[r, pl.ds(j * 16, 16)] = z


def _fill_1d(buf, n, value):
    v = jnp.full((16,), value, jnp.float32)

    @pl.loop(0, n // 16)
    def _(k):
        buf[pl.ds(k * 16, 16)] = v


def _make_part_seg(K, phases, name):
    rb = NR // NSUB
    kp = K // phases
    assert rb % C == 0 and kp * phases == K and kp % 2 == 0

    @functools.partial(
        pl.kernel,
        out_type=jax.ShapeDtypeStruct((NCORE * NR, DIMP), jnp.float32),
        mesh=_MESH,
        scratch_types=[
            pltpu.VMEM((kp, C), jnp.int32),
            pltpu.VMEM((kp, C), jnp.int32),
            pltpu.VMEM((C, DIMP), jnp.float32),
            pltpu.VMEM((C, DIMP), jnp.float32),
            pltpu.VMEM_SHARED((NR, DIMP), jnp.float32),
            pltpu.SemaphoreType.DMA,
            pltpu.SemaphoreType.DMA,
            pltpu.SemaphoreType.DMA,
            pltpu.SemaphoreType.DMA,
        ],
        name=name,
    )
    def kern(table, gidx, sidx, out, gv, sv, r0, r1, acc, sg0, sg1, ss0, ss1):
        cid = lax.axis_index("c")
        sid = lax.axis_index("s")
        w = cid * NSUB + sid

        _fill_zero_rows(r0)
        for k in range(rb // C):
            pltpu.async_copy(r0, acc.at[pl.ds(sid * rb + k * C, C)], sg0)
        for k in range(rb // C):
            pltpu.make_async_copy(
                r0, acc.at[pl.ds(sid * rb + k * C, C)], sg0).wait()
        plsc.subcore_barrier()

        for p in range(phases):
            pltpu.sync_copy(gidx.at[w, pl.ds(p * kp, kp)], gv)
            pltpu.sync_copy(sidx.at[w, pl.ds(p * kp, kp)], sv)
            pltpu.async_copy(table.at[gv.at[0]], r0, sg0)
            pltpu.async_copy(table.at[gv.at[1]], r1, sg1)

            @pl.loop(0, kp, step=2)
            def _(k):
                pltpu.make_async_copy(table.at[pl.ds(0, C)], r0, sg0).wait()
                pltpu.async_copy(r0, acc.at[sv.at[k]], ss0, add=True)
                pltpu.make_async_copy(table.at[pl.ds(0, C)], r1, sg1).wait()
                pltpu.async_copy(r1, acc.at[sv.at[k + 1]], ss1, add=True)
                pltpu.make_async_copy(r0, acc.at[pl.ds(0, C)], ss0).wait()

                @pl.when(k + 2 < kp)
                def _():
                    pltpu.async_copy(table.at[gv.at[k + 2]], r0, sg0)

                pltpu.make_async_copy(r1, acc.at[pl.ds(0, C)], ss1).wait()

                @pl.when(k + 3 < kp)
                def _():
                    pltpu.async_copy(table.at[gv.at[k + 3]], r1, sg1)

        plsc.subcore_barrier()
        for k in range(rb // C):
            off = sid * rb + k * C
            pltpu.async_copy(acc.at[pl.ds(off, C)],
                             out.at[pl.ds(cid * NR + off, C)], sg0)
        for k in range(rb // C):
            off = sid * rb + k * C
            pltpu.make_async_copy(acc.at[pl.ds(off, C)],
                                  out.at[pl.ds(cid * NR + off, C)], sg0).wait()

    return kern


@functools.partial(
    pl.kernel,
    out_type=(
        jax.ShapeDtypeStruct((NCORE * BH, DIMP), jnp.float32),
        jax.ShapeDtypeStruct((NCORE * NR,), jnp.float32),
    ),
    mesh=_MESH,
    scratch_types=[
        pltpu.VMEM((KH, C), jnp.int32),
        pltpu.VMEM((KH, C), jnp.int32),
        pltpu.VMEM((KE, C), jnp.int32),
        pltpu.VMEM((C, DIMP), jnp.float32),
        pltpu.VMEM((C, DIMP), jnp.float32),
        pltpu.VMEM((C,), jnp.float32),
        pltpu.VMEM((C,), jnp.float32),
        pltpu.VMEM_SHARED((BH, DIMP), jnp.float32),
        pltpu.VMEM_SHARED((NR,), jnp.float32),
        pltpu.SemaphoreType.DMA,
        pltpu.SemaphoreType.DMA,
        pltpu.SemaphoreType.DMA,
        pltpu.SemaphoreType.DMA,
        pltpu.SemaphoreType.DMA,
    ],
    name="sc_hyper_fwd_hist",
)
def _sc_hyper_fwd(h, nidx, hidx, didx,
                  esum_o, deg_o,
                  nv, hv, dv, r0, r1, ones, z1,
                  eacc, deg, sg0, sg1, ss0, ss1, sh):
    cid = lax.axis_index("c")
    sid = lax.axis_index("s")
    w = cid * NSUB + sid
    rbh = BH // NSUB
    rbn = NR // NSUB

    _fill_zero_rows(r0)
    _fill_1d(ones, C, 1.0)
    _fill_1d(z1, C, 0.0)
    pltpu.async_copy(r0, eacc.at[pl.ds(sid * rbh, C)], ss0)
    for k in range(rbn // C):
        pltpu.async_copy(z1, deg.at[pl.ds(sid * rbn + k * C, C)], sh)
    pltpu.sync_copy(nidx.at[w], nv)
    pltpu.sync_copy(hidx.at[w], hv)
    pltpu.sync_copy(didx.at[w], dv)
    pltpu.make_async_copy(r0, eacc.at[pl.ds(sid * rbh, C)], ss0).wait()
    for k in range(rbn // C):
        pltpu.make_async_copy(
            z1, deg.at[pl.ds(sid * rbn + k * C, C)], sh).wait()
    plsc.subcore_barrier()

    pltpu.async_copy(h.at[nv.at[0]], r0, sg0)
    pltpu.async_copy(h.at[nv.at[1]], r1, sg1)

    @pl.loop(0, KH, step=2)
    def _(k):
        pltpu.make_async_copy(h.at[pl.ds(0, C)], r0, sg0).wait()
        pltpu.async_copy(r0, eacc.at[hv.at[k]], ss0, add=True)
        pltpu.make_async_copy(h.at[pl.ds(0, C)], r1, sg1).wait()
        pltpu.async_copy(r1, eacc.at[hv.at[k + 1]], ss1, add=True)
        pltpu.make_async_copy(r0, eacc.at[pl.ds(0, C)], ss0).wait()

        @pl.when(k + 2 < KH)
        def _():
            pltpu.async_copy(h.at[nv.at[k + 2]], r0, sg0)

        pltpu.make_async_copy(r1, eacc.at[pl.ds(0, C)], ss1).wait()

        @pl.when(k + 3 < KH)
        def _():
            pltpu.async_copy(h.at[nv.at[k + 3]], r1, sg1)

    @pl.loop(0, KE)
    def _(k):
        pltpu.async_copy(ones, deg.at[dv.at[k]], sh, add=True)

    @pl.loop(0, KE)
    def _(k):
        pltpu.make_async_copy(ones, deg.at[pl.ds(0, C)], sh).wait()

    plsc.subcore_barrier()
    pltpu.async_copy(eacc.at[pl.ds(sid * rbh, C)],
                     esum_o.at[pl.ds(cid * BH + sid * rbh, C)], ss0)
    for k in range(rbn // C):
        off = sid * rbn + k * C
        pltpu.async_copy(deg.at[pl.ds(off, C)],
                         deg_o.at[pl.ds(cid * NR + off, C)], sh)
    pltpu.make_async_copy(eacc.at[pl.ds(sid * rbh, C)],
                          esum_o.at[pl.ds(cid * BH + sid * rbh, C)], ss0).wait()
    for k in range(rbn // C):
        off = sid * rbn + k * C
        pltpu.make_async_copy(deg.at[pl.ds(off, C)],
                              deg_o.at[pl.ds(cid * NR + off, C)], sh).wait()


_seg_vsum = _make_part_seg(KH, 1, "sc_hyper_bwd")
_seg_gcn = _make_part_seg(KE, 2, "sc_gcn_edges")


_RB = 1024


def _row_spec(d):
    return pl.BlockSpec((_RB, d), lambda i: (i, 0))


def _full_spec(a, b):
    return pl.BlockSpec((a, b), lambda i: (0, 0))


def _tc1_body(x_ref, w_ref, b_ref, o_ref):
    o_ref[...] = jax.nn.relu(
        jnp.dot(x_ref[...], w_ref[...], preferred_element_type=jnp.float32)
        + b_ref[...])


def _tc2_body(e0_ref, e1_ref, w_ref, b_ref, o_ref):
    s = e0_ref[...] + e1_ref[...]
    cnt = jnp.maximum(s[:, DIM:DIM + 1], 1.0)
    m = s / cnt
    o_ref[...] = jax.nn.relu(
        jnp.dot(m, w_ref[...], preferred_element_type=jnp.float32) + b_ref[...])


def _tc3_body(v0_ref, v1_ref, dg0_ref, dg1_ref, x_ref,
              wv_ref, bv_ref, w1a_ref, w1b_ref, h1s_ref, dinv_ref):
    s = v0_ref[...] + v1_ref[...]
    m = s / jnp.maximum(s[:, DIM:DIM + 1], 1.0)
    xh = jax.nn.relu(
        jnp.dot(m, wv_ref[...], preferred_element_type=jnp.float32) + bv_ref[...])
    dinv = lax.rsqrt(dg0_ref[...] + dg1_ref[...] + 1.0)
    h1 = (jnp.dot(x_ref[...], w1a_ref[...], preferred_element_type=jnp.float32)
          + jnp.dot(xh, w1b_ref[...], preferred_element_type=jnp.float32))
    h1s_ref[...] = h1 * dinv
    dinv_ref[...] = dinv


def _tc4_body(a0_ref, a1_ref, hs_ref, di_ref, w2_ref, b1_ref, o_ref):
    g1 = jax.nn.relu(
        (a0_ref[...] + a1_ref[...] + hs_ref[...]) * di_ref[...] + b1_ref[...])
    o_ref[...] = jnp.dot(
        g1, w2_ref[...], preferred_element_type=jnp.float32) * di_ref[...]


def _tc5_body(a0_ref, a1_ref, hs_ref, di_ref, b2_ref, wlp_ref, blp_ref, o_ref):
    g2 = (a0_ref[...] + a1_ref[...] + hs_ref[...]) * di_ref[...] + b2_ref[...]
    o_ref[...] = (jnp.dot(g2, wlp_ref[...], preferred_element_type=jnp.float32)
                  + blp_ref[...])


def kernel(x, edge_index, hyperedge_index,
           W_in, b_in, W_e, b_e, W_v, b_v, W1, b1, W2, b2, Wlp, blp):
    f32 = jnp.float32
    src = edge_index[0]
    dst = edge_index[1]
    nidx = hyperedge_index[0]
    hidx = hyperedge_index[1]

    srcf = jnp.concatenate([src, jnp.zeros((EP - E0,), jnp.int32)])
    dstf = jnp.concatenate([dst, jnp.full((EP - E0,), N0, jnp.int32)])
    nidxf = jnp.concatenate([nidx, jnp.full((EHP - EH0,), N0, jnp.int32)])
    hidxf = jnp.concatenate([hidx, jnp.full((EHP - EH0,), NHE0, jnp.int32)])

    nidxp = nidxf.reshape(NW, KH, C)
    hidxp = hidxf.reshape(NW, KH, C)
    dstp = dstf.reshape(NW, KE, C)
    srcp = srcf.reshape(NW, KE, C)

    xp = jnp.zeros((NR, DF), f32).at[:N0].set(x)
    W_inp = jnp.zeros((DF, DIMP), f32).at[:, :DIM].set(W_in)
    b_in2 = jnp.zeros((1, DIMP), f32).at[0, :DIM].set(b_in).at[0, DIM].set(1.0)
    W_ep = jnp.zeros((DIMP, DIMP), f32).at[:DIM, :DIM].set(W_e)
    b_e2 = jnp.zeros((1, DIMP), f32).at[0, :DIM].set(b_e).at[0, DIM].set(1.0)
    W_vp = jnp.zeros((DIMP, DIMP), f32).at[:DIM, :DIM].set(W_v)
    b_v2 = jnp.zeros((1, DIMP), f32).at[0, :DIM].set(b_v)
    W1a = jnp.zeros((DF, DIMP), f32).at[:, :DIM].set(W1[:DF])
    W1b = jnp.zeros((DIMP, DIMP), f32).at[:DIM, :DIM].set(W1[DF:])
    b1_2 = jnp.zeros((1, DIMP), f32).at[0, :DIM].set(b1)
    W2p = jnp.zeros((DIMP, DIMP), f32).at[:DIM, :NCLS].set(W2)
    b2p = jnp.zeros((1, DIMP), f32).at[0, :NCLS].set(b2)
    Wlpp = jnp.zeros((DIMP, NCLS), f32).at[:NCLS].set(Wlp)
    blp2 = blp.reshape(1, NCLS)

    h = pl.pallas_call(
        _tc1_body,
        grid=(NR // _RB,),
        in_specs=[_row_spec(DF), _full_spec(DF, DIMP), _full_spec(1, DIMP)],
        out_specs=_row_spec(DIMP),
        out_shape=jax.ShapeDtypeStruct((NR, DIMP), f32),
    )(xp, W_inp, b_in2)

    esum_p, deg_p = _sc_hyper_fwd(h, nidxp, hidxp, dstp)

    e = pl.pallas_call(
        _tc2_body,
        grid=(1,),
        in_specs=[_full_spec(BH, DIMP), _full_spec(BH, DIMP),
                  _full_spec(DIMP, DIMP), _full_spec(1, DIMP)],
        out_specs=_full_spec(BH, DIMP),
        out_shape=jax.ShapeDtypeStruct((BH, DIMP), f32),
    )(esum_p[:BH], esum_p[BH:], W_ep, b_e2)

    vsum_p = _seg_vsum(e, hidxp, nidxp)

    h1s, dinv = pl.pallas_call(
        _tc3_body,
        grid=(NR // _RB,),
        in_specs=[_row_spec(DIMP), _row_spec(DIMP),
                  _row_spec(1), _row_spec(1),
                  _row_spec(DF), _full_spec(DIMP, DIMP), _full_spec(1, DIMP),
                  _full_spec(DF, DIMP), _full_spec(DIMP, DIMP)],
        out_specs=(_row_spec(DIMP), _row_spec(1)),
        out_shape=(jax.ShapeDtypeStruct((NR, DIMP), f32),
                   jax.ShapeDtypeStruct((NR, 1), f32)),
    )(vsum_p[:NR], vsum_p[NR:],
      deg_p[:NR].reshape(NR, 1), deg_p[NR:].reshape(NR, 1),
      xp, W_vp, b_v2, W1a, W1b)

    acc1_p = _seg_gcn(h1s, srcp, dstp)

    h2s = pl.pallas_call(
        _tc4_body,
        grid=(NR // _RB,),
        in_specs=[_row_spec(DIMP), _row_spec(DIMP), _row_spec(DIMP),
                  _row_spec(1),
                  _full_spec(DIMP, DIMP), _full_spec(1, DIMP)],
        out_specs=_row_spec(DIMP),
        out_shape=jax.ShapeDtypeStruct((NR, DIMP), f32),
    )(acc1_p[:NR], acc1_p[NR:], h1s, dinv, W2p, b1_2)

    acc2_p = _seg_gcn(h2s, srcp, dstp)

    outp = pl.pallas_call(
        _tc5_body,
        grid=(NR // _RB,),
        in_specs=[_row_spec(DIMP), _row_spec(DIMP), _row_spec(DIMP),
                  _row_spec(1),
                  _full_spec(1, DIMP), _full_spec(DIMP, NCLS),
                  _full_spec(1, NCLS)],
        out_specs=_row_spec(NCLS),
        out_shape=jax.ShapeDtypeStruct((NR, NCLS), f32),
    )(acc2_p[:NR], acc2_p[NR:], h2s, dinv, b2p, Wlpp, blp2)

    return outp[:N0]


# speedup vs baseline: 1.4268x; 1.4268x over previous
import functools

import jax
import jax.numpy as jnp
from jax import lax
from jax.experimental import pallas as pl
from jax.experimental.pallas import tpu as pltpu
from jax.experimental.pallas import tpu_sc as plsc

N0 = 10000
E0 = 320000
EH0 = 100000
NHE0 = 2000
DF = 128
DIM = 64
DIMP = 128
NCLS = 40

NCORE = 2
NSUB = 16
NW = NCORE * NSUB
C = 128

NR = 10240
BH = 2048

KE = 80
KH = 26
EP = NW * KE * C
EHP = NW * KH * C

_MESH = plsc.VectorSubcoreMesh(
    core_axis_name="c", subcore_axis_name="s",
    num_cores=NCORE, num_subcores=NSUB)


def _fill_zero_rows(buf):
    z = jnp.zeros((16,), jnp.float32)

    @pl.loop(0, C)
    def _(r):
        for j in range(DIMP // 16):
            buf[r, pl.ds(j * 16, 16)] = z


def _fill_1d(buf, n, value):
    v = jnp.full((16,), value, jnp.float32)

    @pl.loop(0, n // 16)
    def _(k):
        buf[pl.ds(k * 16, 16)] = v


def _make_part_seg(K, phases, name):
    rb = NR // NSUB
    kp = K // phases
    assert rb % C == 0 and kp * phases == K and kp % 2 == 0

    @functools.partial(
        pl.kernel,
        out_type=jax.ShapeDtypeStruct((NCORE * NR, DIMP), jnp.float32),
        mesh=_MESH,
        scratch_types=[
            pltpu.VMEM((kp, C), jnp.int32),
            pltpu.VMEM((kp, C), jnp.int32),
            pltpu.VMEM((C, DIMP), jnp.float32),
            pltpu.VMEM((C, DIMP), jnp.float32),
            pltpu.VMEM_SHARED((NR, DIMP), jnp.float32),
            pltpu.SemaphoreType.DMA,
            pltpu.SemaphoreType.DMA,
            pltpu.SemaphoreType.DMA,
            pltpu.SemaphoreType.DMA,
        ],
        name=name,
    )
    def kern(table, gidx, sidx, out, gv, sv, r0, r1, acc, sg0, sg1, ss0, ss1):
        cid = lax.axis_index("c")
        sid = lax.axis_index("s")
        w = cid * NSUB + sid

        _fill_zero_rows(r0)
        for k in range(rb // C):
            pltpu.async_copy(r0, acc.at[pl.ds(sid * rb + k * C, C)], sg0)
        for k in range(rb // C):
            pltpu.make_async_copy(
                r0, acc.at[pl.ds(sid * rb + k * C, C)], sg0).wait()
        plsc.subcore_barrier()

        for p in range(phases):
            pltpu.sync_copy(gidx.at[w, pl.ds(p * kp, kp)], gv)
            pltpu.sync_copy(sidx.at[w, pl.ds(p * kp, kp)], sv)
            pltpu.async_copy(table.at[gv.at[0]], r0, sg0)
            pltpu.async_copy(table.at[gv.at[1]], r1, sg1)

            @pl.loop(0, kp, step=2)
            def _(k):
                pltpu.make_async_copy(table.at[pl.ds(0, C)], r0, sg0).wait()
                pltpu.async_copy(r0, acc.at[sv.at[k]], ss0, add=True)
                pltpu.make_async_copy(table.at[pl.ds(0, C)], r1, sg1).wait()
                pltpu.async_copy(r1, acc.at[sv.at[k + 1]], ss1, add=True)
                pltpu.make_async_copy(r0, acc.at[pl.ds(0, C)], ss0).wait()

                @pl.when(k + 2 < kp)
                def _():
                    pltpu.async_copy(table.at[gv.at[k + 2]], r0, sg0)

                pltpu.make_async_copy(r1, acc.at[pl.ds(0, C)], ss1).wait()

                @pl.when(k + 3 < kp)
                def _():
                    pltpu.async_copy(table.at[gv.at[k + 3]], r1, sg1)

        plsc.subcore_barrier()
        for k in range(rb // C):
            off = sid * rb + k * C
            pltpu.async_copy(acc.at[pl.ds(off, C)],
                             out.at[pl.ds(cid * NR + off, C)], sg0)
        for k in range(rb // C):
            off = sid * rb + k * C
            pltpu.make_async_copy(acc.at[pl.ds(off, C)],
                                  out.at[pl.ds(cid * NR + off, C)], sg0).wait()

    return kern


@functools.partial(
    pl.kernel,
    out_type=(
        jax.ShapeDtypeStruct((NCORE * BH, DIMP), jnp.float32),
        jax.ShapeDtypeStruct((NCORE * NR,), jnp.float32),
    ),
    mesh=_MESH,
    scratch_types=[
        pltpu.VMEM((KH, C), jnp.int32),
        pltpu.VMEM((KH, C), jnp.int32),
        pltpu.VMEM((KE, C), jnp.int32),
        pltpu.VMEM((C, DIMP), jnp.float32),
        pltpu.VMEM((C, DIMP), jnp.float32),
        pltpu.VMEM((C,), jnp.float32),
        pltpu.VMEM((C,), jnp.float32),
        pltpu.VMEM_SHARED((BH, DIMP), jnp.float32),
        pltpu.VMEM_SHARED((NR,), jnp.float32),
        pltpu.SemaphoreType.DMA,
        pltpu.SemaphoreType.DMA,
        pltpu.SemaphoreType.DMA,
        pltpu.SemaphoreType.DMA,
        pltpu.SemaphoreType.DMA,
    ],
    name="sc_hyper_fwd_hist",
)
def _sc_hyper_fwd(h, nidx, hidx, didx,
                  esum_o, deg_o,
                  nv, hv, dv, r0, r1, ones, z1,
                  eacc, deg, sg0, sg1, ss0, ss1, sh):
    cid = lax.axis_index("c")
    sid = lax.axis_index("s")
    w = cid * NSUB + sid
    rbh = BH // NSUB
    rbn = NR // NSUB

    _fill_zero_rows(r0)
    _fill_1d(ones, C, 1.0)
    _fill_1d(z1, C, 0.0)
    pltpu.async_copy(r0, eacc.at[pl.ds(sid * rbh, C)], ss0)
    for k in range(rbn // C):
        pltpu.async_copy(z1, deg.at[pl.ds(sid * rbn + k * C, C)], sh)
    pltpu.sync_copy(nidx.at[w], nv)
    pltpu.sync_copy(hidx.at[w], hv)
    pltpu.sync_copy(didx.at[w], dv)
    pltpu.make_async_copy(r0, eacc.at[pl.ds(sid * rbh, C)], ss0).wait()
    for k in range(rbn // C):
        pltpu.make_async_copy(
            z1, deg.at[pl.ds(sid * rbn + k * C, C)], sh).wait()
    plsc.subcore_barrier()

    pltpu.async_copy(h.at[nv.at[0]], r0, sg0)
    pltpu.async_copy(h.at[nv.at[1]], r1, sg1)

    @pl.loop(0, KH, step=2)
    def _(k):
        pltpu.make_async_copy(h.at[pl.ds(0, C)], r0, sg0).wait()
        pltpu.async_copy(r0, eacc.at[hv.at[k]], ss0, add=True)
        pltpu.make_async_copy(h.at[pl.ds(0, C)], r1, sg1).wait()
        pltpu.async_copy(r1, eacc.at[hv.at[k + 1]], ss1, add=True)
        pltpu.make_async_copy(r0, eacc.at[pl.ds(0, C)], ss0).wait()

        @pl.when(k + 2 < KH)
        def _():
            pltpu.async_copy(h.at[nv.at[k + 2]], r0, sg0)

        pltpu.make_async_copy(r1, eacc.at[pl.ds(0, C)], ss1).wait()

        @pl.when(k + 3 < KH)
        def _():
            pltpu.async_copy(h.at[nv.at[k + 3]], r1, sg1)

    @pl.loop(0, KE)
    def _(k):
        pltpu.async_copy(ones, deg.at[dv.at[k]], sh, add=True)

    @pl.loop(0, KE)
    def _(k):
        pltpu.make_async_copy(ones, deg.at[pl.ds(0, C)], sh).wait()

    plsc.subcore_barrier()
    pltpu.async_copy(eacc.at[pl.ds(sid * rbh, C)],
                     esum_o.at[pl.ds(cid * BH + sid * rbh, C)], ss0)
    for k in range(rbn // C):
        off = sid * rbn + k * C
        pltpu.async_copy(deg.at[pl.ds(off, C)],
                         deg_o.at[pl.ds(cid * NR + off, C)], sh)
    pltpu.make_async_copy(eacc.at[pl.ds(sid * rbh, C)],
                          esum_o.at[pl.ds(cid * BH + sid * rbh, C)], ss0).wait()
    for k in range(rbn // C):
        off = sid * rbn + k * C
        pltpu.make_async_copy(deg.at[pl.ds(off, C)],
                              deg_o.at[pl.ds(cid * NR + off, C)], sh).wait()


_seg_vsum = _make_part_seg(KH, 1, "sc_hyper_bwd")
_seg_gcn = _make_part_seg(KE, 2, "sc_gcn_edges")


_RB = 1024


def _row_spec(d):
    return pl.BlockSpec((_RB, d), lambda i: (i, 0))


def _full_spec(a, b):
    return pl.BlockSpec((a, b), lambda i: (0, 0))


def _tc1_body(x_ref, w_ref, b_ref, o_ref):
    o_ref[...] = jax.nn.relu(
        jnp.dot(x_ref[...], w_ref[...], preferred_element_type=jnp.float32)
        + b_ref[...])


def _tc2_body(e0_ref, e1_ref, w_ref, b_ref, o_ref):
    s = e0_ref[...] + e1_ref[...]
    cnt = jnp.maximum(s[:, DIM:DIM + 1], 1.0)
    m = s / cnt
    o_ref[...] = jax.nn.relu(
        jnp.dot(m, w_ref[...], preferred_element_type=jnp.float32) + b_ref[...])


def _tc3_body(v0_ref, v1_ref, dg0_ref, dg1_ref, x_ref,
              wv_ref, bv_ref, w1a_ref, w1b_ref, h1s_ref, dinv_ref):
    s = v0_ref[...] + v1_ref[...]
    m = s / jnp.maximum(s[:, DIM:DIM + 1], 1.0)
    xh = jax.nn.relu(
        jnp.dot(m, wv_ref[...], preferred_element_type=jnp.float32) + bv_ref[...])
    dinv = lax.rsqrt(dg0_ref[...] + dg1_ref[...] + 1.0)
    h1 = (jnp.dot(x_ref[...], w1a_ref[...], preferred_element_type=jnp.float32)
          + jnp.dot(xh, w1b_ref[...], preferred_element_type=jnp.float32))
    h1s_ref[...] = h1 * dinv
    dinv_ref[...] = dinv


def _tc4_body(a0_ref, a1_ref, hs_ref, di_ref, w2_ref, b1_ref, o_ref):
    g1 = jax.nn.relu(
        (a0_ref[...] + a1_ref[...] + hs_ref[...]) * di_ref[...] + b1_ref[...])
    o_ref[...] = jnp.dot(
        g1, w2_ref[...], preferred_element_type=jnp.float32) * di_ref[...]


def _tc5_body(a0_ref, a1_ref, hs_ref, di_ref, b2_ref, wlp_ref, blp_ref, o_ref):
    g2 = (a0_ref[...] + a1_ref[...] + hs_ref[...]) * di_ref[...] + b2_ref[...]
    o_ref[...] = (jnp.dot(g2, wlp_ref[...], preferred_element_type=jnp.float32)
                  + blp_ref[...])


def kernel(x, edge_index, hyperedge_index,
           W_in, b_in, W_e, b_e, W_v, b_v, W1, b1, W2, b2, Wlp, blp):
    f32 = jnp.float32
    src = edge_index[0]
    dst = edge_index[1]
    nidx = hyperedge_index[0]
    hidx = hyperedge_index[1]

    tail_e = N0 + (jnp.arange(EP - E0, dtype=jnp.int32) % (NR - N0))
    tail_n = N0 + (jnp.arange(EHP - EH0, dtype=jnp.int32) % (NR - N0))
    tail_h = NHE0 + (jnp.arange(EHP - EH0, dtype=jnp.int32) % (BH - NHE0))
    srcf = jnp.concatenate([src, jnp.zeros((EP - E0,), jnp.int32)])
    dstf = jnp.concatenate([dst, tail_e])
    nidxf = jnp.concatenate([nidx, tail_n])
    hidxf = jnp.concatenate([hidx, tail_h])

    nidxp = nidxf.reshape(NW, KH, C)
    hidxp = hidxf.reshape(NW, KH, C)
    dstp = dstf.reshape(NW, KE, C)
    srcp = srcf.reshape(NW, KE, C)

    xp = jnp.zeros((NR, DF), f32).at[:N0].set(x)
    W_inp = jnp.zeros((DF, DIMP), f32).at[:, :DIM].set(W_in)
    b_in2 = jnp.zeros((1, DIMP), f32).at[0, :DIM].set(b_in).at[0, DIM].set(1.0)
    W_ep = jnp.zeros((DIMP, DIMP), f32).at[:DIM, :DIM].set(W_e)
    b_e2 = jnp.zeros((1, DIMP), f32).at[0, :DIM].set(b_e).at[0, DIM].set(1.0)
    W_vp = jnp.zeros((DIMP, DIMP), f32).at[:DIM, :DIM].set(W_v)
    b_v2 = jnp.zeros((1, DIMP), f32).at[0, :DIM].set(b_v)
    W1a = jnp.zeros((DF, DIMP), f32).at[:, :DIM].set(W1[:DF])
    W1b = jnp.zeros((DIMP, DIMP), f32).at[:DIM, :DIM].set(W1[DF:])
    b1_2 = jnp.zeros((1, DIMP), f32).at[0, :DIM].set(b1)
    W2p = jnp.zeros((DIMP, DIMP), f32).at[:DIM, :NCLS].set(W2)
    b2p = jnp.zeros((1, DIMP), f32).at[0, :NCLS].set(b2)
    Wlpp = jnp.zeros((DIMP, NCLS), f32).at[:NCLS].set(Wlp)
    blp2 = blp.reshape(1, NCLS)

    h = pl.pallas_call(
        _tc1_body,
        grid=(NR // _RB,),
        in_specs=[_row_spec(DF), _full_spec(DF, DIMP), _full_spec(1, DIMP)],
        out_specs=_row_spec(DIMP),
        out_shape=jax.ShapeDtypeStruct((NR, DIMP), f32),
    )(xp, W_inp, b_in2)

    esum_p, deg_p = _sc_hyper_fwd(h, nidxp, hidxp, dstp)

    e = pl.pallas_call(
        _tc2_body,
        grid=(1,),
        in_specs=[_full_spec(BH, DIMP), _full_spec(BH, DIMP),
                  _full_spec(DIMP, DIMP), _full_spec(1, DIMP)],
        out_specs=_full_spec(BH, DIMP),
        out_shape=jax.ShapeDtypeStruct((BH, DIMP), f32),
    )(esum_p[:BH], esum_p[BH:], W_ep, b_e2)

    vsum_p = _seg_vsum(e, hidxp, nidxp)

    h1s, dinv = pl.pallas_call(
        _tc3_body,
        grid=(NR // _RB,),
        in_specs=[_row_spec(DIMP), _row_spec(DIMP),
                  _row_spec(1), _row_spec(1),
                  _row_spec(DF), _full_spec(DIMP, DIMP), _full_spec(1, DIMP),
                  _full_spec(DF, DIMP), _full_spec(DIMP, DIMP)],
        out_specs=(_row_spec(DIMP), _row_spec(1)),
        out_shape=(jax.ShapeDtypeStruct((NR, DIMP), f32),
                   jax.ShapeDtypeStruct((NR, 1), f32)),
    )(vsum_p[:NR], vsum_p[NR:],
      deg_p[:NR].reshape(NR, 1), deg_p[NR:].reshape(NR, 1),
      xp, W_vp, b_v2, W1a, W1b)

    acc1_p = _seg_gcn(h1s, srcp, dstp)

    h2s = pl.pallas_call(
        _tc4_body,
        grid=(NR // _RB,),
        in_specs=[_row_spec(DIMP), _row_spec(DIMP), _row_spec(DIMP),
                  _row_spec(1),
                  _full_spec(DIMP, DIMP), _full_spec(1, DIMP)],
        out_specs=_row_spec(DIMP),
        out_shape=jax.ShapeDtypeStruct((NR, DIMP), f32),
    )(acc1_p[:NR], acc1_p[NR:], h1s, dinv, W2p, b1_2)

    acc2_p = _seg_gcn(h2s, srcp, dstp)

    outp = pl.pallas_call(
        _tc5_body,
        grid=(NR // _RB,),
        in_specs=[_row_spec(DIMP), _row_spec(DIMP), _row_spec(DIMP),
                  _row_spec(1),
                  _full_spec(1, DIMP), _full_spec(DIMP, NCLS),
                  _full_spec(1, NCLS)],
        out_specs=_row_spec(NCLS),
        out_shape=jax.ShapeDtypeStruct((NR, NCLS), f32),
    )(acc2_p[:NR], acc2_p[NR:], h2s, dinv, b2p, Wlpp, blp2)

    return outp[:N0]


# speedup vs baseline: 3.4111x; 2.3908x over previous
import functools

import jax
import jax.numpy as jnp
from jax import lax
from jax.experimental import pallas as pl
from jax.experimental.pallas import tpu as pltpu
from jax.experimental.pallas import tpu_sc as plsc

N0 = 10000
E0 = 320000
EH0 = 100000
NHE0 = 2000
DF = 128
DIM = 64
DIMP = 128
NCLS = 40

NCORE = 2
NSUB = 16
NW = NCORE * NSUB
C = 128

NR = 10240
BH = 2048

KE = 80
KH = 26
EP = NW * KE * C
EHP = NW * KH * C

_MESH = plsc.VectorSubcoreMesh(
    core_axis_name="c", subcore_axis_name="s",
    num_cores=NCORE, num_subcores=NSUB)


def _fill_zero_rows(buf):
    z = jnp.zeros((16,), jnp.float32)

    @pl.loop(0, C)
    def _(r):
        for j in range(DIMP // 16):
            buf[r, pl.ds(j * 16, 16)] = z


def _fill_1d(buf, n, value):
    v = jnp.full((16,), value, jnp.float32)

    @pl.loop(0, n // 16)
    def _(k):
        buf[pl.ds(k * 16, 16)] = v


def _make_part_seg(K, phases, name):
    rb = NR // NSUB
    kp = K // phases
    assert rb % C == 0 and kp * phases == K and kp % 2 == 0

    @functools.partial(
        pl.kernel,
        out_type=jax.ShapeDtypeStruct((NCORE * NR, DIMP), jnp.float32),
        mesh=_MESH,
        scratch_types=[
            pltpu.VMEM((kp, C), jnp.int32),
            pltpu.VMEM((kp, C), jnp.int32),
            pltpu.VMEM((C, DIMP), jnp.float32),
            pltpu.VMEM((C, DIMP), jnp.float32),
            pltpu.VMEM_SHARED((NR, DIMP), jnp.float32),
            pltpu.SemaphoreType.DMA,
            pltpu.SemaphoreType.DMA,
            pltpu.SemaphoreType.DMA,
            pltpu.SemaphoreType.DMA,
        ],
        name=name,
    )
    def kern(table, gidx, sidx, out, gv, sv, r0, r1, acc, sg0, sg1, ss0, ss1):
        cid = lax.axis_index("c")
        sid = lax.axis_index("s")
        w = cid * NSUB + sid

        _fill_zero_rows(r0)
        for k in range(rb // C):
            pltpu.async_copy(r0, acc.at[pl.ds(sid * rb + k * C, C)], sg0)
        for k in range(rb // C):
            pltpu.make_async_copy(
                r0, acc.at[pl.ds(sid * rb + k * C, C)], sg0).wait()
        plsc.subcore_barrier()

        for p in range(phases):
            pltpu.sync_copy(gidx.at[w, pl.ds(p * kp, kp)], gv)
            pltpu.sync_copy(sidx.at[w, pl.ds(p * kp, kp)], sv)
            pltpu.async_copy(table.at[gv.at[0]], r0, sg0)
            pltpu.async_copy(table.at[gv.at[1]], r1, sg1)

            @pl.loop(0, kp, step=2)
            def _(k):
                pltpu.make_async_copy(table.at[pl.ds(0, C)], r0, sg0).wait()
                pltpu.async_copy(r0, acc.at[sv.at[k]], ss0, add=True)
                pltpu.make_async_copy(table.at[pl.ds(0, C)], r1, sg1).wait()
                pltpu.async_copy(r1, acc.at[sv.at[k + 1]], ss1, add=True)
                pltpu.make_async_copy(r0, acc.at[pl.ds(0, C)], ss0).wait()

                @pl.when(k + 2 < kp)
                def _():
                    pltpu.async_copy(table.at[gv.at[k + 2]], r0, sg0)

                pltpu.make_async_copy(r1, acc.at[pl.ds(0, C)], ss1).wait()

                @pl.when(k + 3 < kp)
                def _():
                    pltpu.async_copy(table.at[gv.at[k + 3]], r1, sg1)

        plsc.subcore_barrier()
        for k in range(rb // C):
            off = sid * rb + k * C
            pltpu.async_copy(acc.at[pl.ds(off, C)],
                             out.at[pl.ds(cid * NR + off, C)], sg0)
        for k in range(rb // C):
            off = sid * rb + k * C
            pltpu.make_async_copy(acc.at[pl.ds(off, C)],
                                  out.at[pl.ds(cid * NR + off, C)], sg0).wait()

    return kern


@functools.partial(
    pl.kernel,
    out_type=(
        jax.ShapeDtypeStruct((NCORE * BH, DIMP), jnp.float32),
        jax.ShapeDtypeStruct((NCORE * NR,), jnp.float32),
    ),
    mesh=_MESH,
    scratch_types=[
        pltpu.VMEM((KH, C), jnp.int32),
        pltpu.VMEM((KH, C), jnp.int32),
        pltpu.VMEM((KE, C), jnp.int32),
        pltpu.VMEM((C, DIMP), jnp.float32),
        pltpu.VMEM((C, DIMP), jnp.float32),
        pltpu.VMEM((C,), jnp.float32),
        pltpu.VMEM((C,), jnp.float32),
        pltpu.VMEM_SHARED((BH, DIMP), jnp.float32),
        pltpu.VMEM_SHARED((NR,), jnp.float32),
        pltpu.SemaphoreType.DMA,
        pltpu.SemaphoreType.DMA,
        pltpu.SemaphoreType.DMA,
        pltpu.SemaphoreType.DMA,
        pltpu.SemaphoreType.DMA,
    ],
    name="sc_hyper_fwd_hist",
)
def _sc_hyper_fwd(h, nidx, hidx, didx,
                  esum_o, deg_o,
                  nv, hv, dv, r0, r1, ones, z1,
                  eacc, deg, sg0, sg1, ss0, ss1, sh):
    cid = lax.axis_index("c")
    sid = lax.axis_index("s")
    w = cid * NSUB + sid
    rbh = BH // NSUB
    rbn = NR // NSUB

    _fill_zero_rows(r0)
    _fill_1d(ones, C, 1.0)
    _fill_1d(z1, C, 0.0)
    pltpu.async_copy(r0, eacc.at[pl.ds(sid * rbh, C)], ss0)
    for k in range(rbn // C):
        pltpu.async_copy(z1, deg.at[pl.ds(sid * rbn + k * C, C)], sh)
    pltpu.sync_copy(nidx.at[w], nv)
    pltpu.sync_copy(hidx.at[w], hv)
    pltpu.sync_copy(didx.at[w], dv)
    pltpu.make_async_copy(r0, eacc.at[pl.ds(sid * rbh, C)], ss0).wait()
    for k in range(rbn // C):
        pltpu.make_async_copy(
            z1, deg.at[pl.ds(sid * rbn + k * C, C)], sh).wait()
    plsc.subcore_barrier()

    pltpu.async_copy(h.at[nv.at[0]], r0, sg0)
    pltpu.async_copy(h.at[nv.at[1]], r1, sg1)

    @pl.loop(0, KH, step=2)
    def _(k):
        pltpu.make_async_copy(h.at[pl.ds(0, C)], r0, sg0).wait()
        pltpu.async_copy(r0, eacc.at[hv.at[k]], ss0, add=True)
        pltpu.make_async_copy(h.at[pl.ds(0, C)], r1, sg1).wait()
        pltpu.async_copy(r1, eacc.at[hv.at[k + 1]], ss1, add=True)
        pltpu.make_async_copy(r0, eacc.at[pl.ds(0, C)], ss0).wait()

        @pl.when(k + 2 < KH)
        def _():
            pltpu.async_copy(h.at[nv.at[k + 2]], r0, sg0)

        pltpu.make_async_copy(r1, eacc.at[pl.ds(0, C)], ss1).wait()

        @pl.when(k + 3 < KH)
        def _():
            pltpu.async_copy(h.at[nv.at[k + 3]], r1, sg1)

    @pl.loop(0, KE)
    def _(k):
        pltpu.async_copy(ones, deg.at[dv.at[k]], sh, add=True)

    @pl.loop(0, KE)
    def _(k):
        pltpu.make_async_copy(ones, deg.at[pl.ds(0, C)], sh).wait()

    plsc.subcore_barrier()
    pltpu.async_copy(eacc.at[pl.ds(sid * rbh, C)],
                     esum_o.at[pl.ds(cid * BH + sid * rbh, C)], ss0)
    for k in range(rbn // C):
        off = sid * rbn + k * C
        pltpu.async_copy(deg.at[pl.ds(off, C)],
                         deg_o.at[pl.ds(cid * NR + off, C)], sh)
    pltpu.make_async_copy(eacc.at[pl.ds(sid * rbh, C)],
                          esum_o.at[pl.ds(cid * BH + sid * rbh, C)], ss0).wait()
    for k in range(rbn // C):
        off = sid * rbn + k * C
        pltpu.make_async_copy(deg.at[pl.ds(off, C)],
                              deg_o.at[pl.ds(cid * NR + off, C)], sh).wait()


_seg_vsum = _make_part_seg(KH, 1, "sc_hyper_bwd")
_seg_gcn = _make_part_seg(KE, 2, "sc_gcn_edges")


_RB = 1024


def _row_spec(d):
    return pl.BlockSpec((_RB, d), lambda i: (i, 0))


def _full_spec(a, b):
    return pl.BlockSpec((a, b), lambda i: (0, 0))


def _tc1_body(x_ref, w_ref, b_ref, o_ref):
    o_ref[...] = jax.nn.relu(
        jnp.dot(x_ref[...], w_ref[...], preferred_element_type=jnp.float32)
        + b_ref[...])


def _tc2_body(e0_ref, e1_ref, w_ref, b_ref, o_ref):
    s = e0_ref[...] + e1_ref[...]
    cnt = jnp.maximum(s[:, DIM:DIM + 1], 1.0)
    m = s / cnt
    o_ref[...] = jax.nn.relu(
        jnp.dot(m, w_ref[...], preferred_element_type=jnp.float32) + b_ref[...])


def _tc3_body(v0_ref, v1_ref, dg0_ref, dg1_ref, x_ref,
              wv_ref, bv_ref, w1a_ref, w1b_ref, h1s_ref, dinv_ref):
    s = v0_ref[...] + v1_ref[...]
    m = s / jnp.maximum(s[:, DIM:DIM + 1], 1.0)
    xh = jax.nn.relu(
        jnp.dot(m, wv_ref[...], preferred_element_type=jnp.float32) + bv_ref[...])
    dinv = lax.rsqrt(dg0_ref[...] + dg1_ref[...] + 1.0)
    h1 = (jnp.dot(x_ref[...], w1a_ref[...], preferred_element_type=jnp.float32)
          + jnp.dot(xh, w1b_ref[...], preferred_element_type=jnp.float32))
    h1s_ref[...] = h1 * dinv
    dinv_ref[...] = dinv


def _tc4_body(a0_ref, a1_ref, hs_ref, di_ref, w2_ref, b1_ref, o_ref):
    g1 = jax.nn.relu(
        (a0_ref[...] + a1_ref[...] + hs_ref[...]) * di_ref[...] + b1_ref[...])
    o_ref[...] = jnp.dot(
        g1, w2_ref[...], preferred_element_type=jnp.float32) * di_ref[...]


def _tc5_body(a0_ref, a1_ref, hs_ref, di_ref, b2_ref, wlp_ref, blp_ref, o_ref):
    g2 = (a0_ref[...] + a1_ref[...] + hs_ref[...]) * di_ref[...] + b2_ref[...]
    o_ref[...] = (jnp.dot(g2, wlp_ref[...], preferred_element_type=jnp.float32)
                  + blp_ref[...])


def kernel(x, edge_index, hyperedge_index,
           W_in, b_in, W_e, b_e, W_v, b_v, W1, b1, W2, b2, Wlp, blp):
    f32 = jnp.float32
    src = edge_index[0]
    dst = edge_index[1]
    nidx = hyperedge_index[0]
    hidx = hyperedge_index[1]

    tail_e = N0 + (jnp.arange(EP - E0, dtype=jnp.int32) % (NR - N0))
    tail_n = N0 + (jnp.arange(EHP - EH0, dtype=jnp.int32) % (NR - N0))
    tail_h = NHE0 + (jnp.arange(EHP - EH0, dtype=jnp.int32) % (BH - NHE0))
    tail_s = jnp.arange(EP - E0, dtype=jnp.int32) % N0
    srcf = jnp.concatenate([src, tail_s])
    dstf = jnp.concatenate([dst, tail_e])
    nidxf = jnp.concatenate([nidx, tail_n])
    hidxf = jnp.concatenate([hidx, tail_h])

    nidxp = nidxf.reshape(NW, KH, C)
    hidxp = hidxf.reshape(NW, KH, C)
    dstp = dstf.reshape(NW, KE, C)
    srcp = srcf.reshape(NW, KE, C)

    xp = jnp.zeros((NR, DF), f32).at[:N0].set(x)
    W_inp = jnp.zeros((DF, DIMP), f32).at[:, :DIM].set(W_in)
    b_in2 = jnp.zeros((1, DIMP), f32).at[0, :DIM].set(b_in).at[0, DIM].set(1.0)
    W_ep = jnp.zeros((DIMP, DIMP), f32).at[:DIM, :DIM].set(W_e)
    b_e2 = jnp.zeros((1, DIMP), f32).at[0, :DIM].set(b_e).at[0, DIM].set(1.0)
    W_vp = jnp.zeros((DIMP, DIMP), f32).at[:DIM, :DIM].set(W_v)
    b_v2 = jnp.zeros((1, DIMP), f32).at[0, :DIM].set(b_v)
    W1a = jnp.zeros((DF, DIMP), f32).at[:, :DIM].set(W1[:DF])
    W1b = jnp.zeros((DIMP, DIMP), f32).at[:DIM, :DIM].set(W1[DF:])
    b1_2 = jnp.zeros((1, DIMP), f32).at[0, :DIM].set(b1)
    W2p = jnp.zeros((DIMP, DIMP), f32).at[:DIM, :NCLS].set(W2)
    b2p = jnp.zeros((1, DIMP), f32).at[0, :NCLS].set(b2)
    Wlpp = jnp.zeros((DIMP, NCLS), f32).at[:NCLS].set(Wlp)
    blp2 = blp.reshape(1, NCLS)

    h = pl.pallas_call(
        _tc1_body,
        grid=(NR // _RB,),
        in_specs=[_row_spec(DF), _full_spec(DF, DIMP), _full_spec(1, DIMP)],
        out_specs=_row_spec(DIMP),
        out_shape=jax.ShapeDtypeStruct((NR, DIMP), f32),
    )(xp, W_inp, b_in2)

    esum_p, deg_p = _sc_hyper_fwd(h, nidxp, hidxp, dstp)

    e = pl.pallas_call(
        _tc2_body,
        grid=(1,),
        in_specs=[_full_spec(BH, DIMP), _full_spec(BH, DIMP),
                  _full_spec(DIMP, DIMP), _full_spec(1, DIMP)],
        out_specs=_full_spec(BH, DIMP),
        out_shape=jax.ShapeDtypeStruct((BH, DIMP), f32),
    )(esum_p[:BH], esum_p[BH:], W_ep, b_e2)

    vsum_p = _seg_vsum(e, hidxp, nidxp)

    h1s, dinv = pl.pallas_call(
        _tc3_body,
        grid=(NR // _RB,),
        in_specs=[_row_spec(DIMP), _row_spec(DIMP),
                  _row_spec(1), _row_spec(1),
                  _row_spec(DF), _full_spec(DIMP, DIMP), _full_spec(1, DIMP),
                  _full_spec(DF, DIMP), _full_spec(DIMP, DIMP)],
        out_specs=(_row_spec(DIMP), _row_spec(1)),
        out_shape=(jax.ShapeDtypeStruct((NR, DIMP), f32),
                   jax.ShapeDtypeStruct((NR, 1), f32)),
    )(vsum_p[:NR], vsum_p[NR:],
      deg_p[:NR].reshape(NR, 1), deg_p[NR:].reshape(NR, 1),
      xp, W_vp, b_v2, W1a, W1b)

    acc1_p = _seg_gcn(h1s, srcp, dstp)

    h2s = pl.pallas_call(
        _tc4_body,
        grid=(NR // _RB,),
        in_specs=[_row_spec(DIMP), _row_spec(DIMP), _row_spec(DIMP),
                  _row_spec(1),
                  _full_spec(DIMP, DIMP), _full_spec(1, DIMP)],
        out_specs=_row_spec(DIMP),
        out_shape=jax.ShapeDtypeStruct((NR, DIMP), f32),
    )(acc1_p[:NR], acc1_p[NR:], h1s, dinv, W2p, b1_2)

    acc2_p = _seg_gcn(h2s, srcp, dstp)

    outp = pl.pallas_call(
        _tc5_body,
        grid=(NR // _RB,),
        in_specs=[_row_spec(DIMP), _row_spec(DIMP), _row_spec(DIMP),
                  _row_spec(1),
                  _full_spec(1, DIMP), _full_spec(DIMP, NCLS),
                  _full_spec(1, NCLS)],
        out_specs=_row_spec(NCLS),
        out_shape=jax.ShapeDtypeStruct((NR, NCLS), f32),
    )(acc2_p[:NR], acc2_p[NR:], h2s, dinv, b2p, Wlpp, blp2)

    return outp[:N0]
